# Initial kernel scaffold; baseline (speedup 1.0000x reference)
import jax
import jax.numpy as jnp
from jax.experimental import pallas as pl


def kernel(x, W1, W2, W3, Wgcn, lin_w, lin_b):
    raise NotImplementedError("write your pallas kernel here")


# speedup vs baseline: 2.5849x; 2.5849x over previous
import jax
import jax.numpy as jnp
from jax.experimental import pallas as pl

N = 1024
IN_CH = 512
OUT_CH = 512
NUM_CHANNELS = 2


def _softmax_col(col):
    m = jnp.max(col, axis=0, keepdims=True)
    e = jnp.exp(col - m)
    return e / jnp.sum(e, axis=0, keepdims=True)


def _body(x_ref, w1t_ref, w2t_ref, w3t_ref, wgcn_ref, linw_ref, linb_ref,
          out_ref):
    x = x_ref[...]
    Xw = jnp.dot(x, wgcn_ref[...], preferred_element_type=jnp.float32)

    acc = None
    for c in range(NUM_CHANNELS):
        f = _softmax_col(w1t_ref[:, c:c + 1])
        g = _softmax_col(w2t_ref[:, c:c + 1])
        h = _softmax_col(w3t_ref[:, c:c + 1])
        S_g = jnp.sum(g, axis=0, keepdims=True)
        S_f = jnp.sum(f, axis=0, keepdims=True)
        u = S_g * f - f * g
        S_u = jnp.sum(u, axis=0, keepdims=True)
        d = (S_u - u) - g * (S_f - f)
        inv_d = jnp.where(d == 0.0, 0.0, 1.0 / d)
        r = h * inv_d
        w = g * r
        R_ = jnp.sum(r, axis=0, keepdims=True)
        Wt = jnp.sum(w, axis=0, keepdims=True)
        v = u * (R_ - r) - f * (Wt - w)
        S_v = jnp.sum(v, axis=0, keepdims=True)
        deg2 = 1.0 + (S_v - v) - r * (S_u - u) + w * (S_f - f)
        inv2 = jnp.where(deg2 == 0.0, 0.0, 1.0 / deg2)
        alpha = 1.0 - v + r * u - w * f

        Sv_vec = jnp.sum(v * Xw, axis=0, keepdims=True)
        Su_vec = jnp.sum(u * Xw, axis=0, keepdims=True)
        Sf_vec = jnp.sum(f * Xw, axis=0, keepdims=True)

        out_c = jnp.maximum(
            inv2 * (alpha * Xw + Sv_vec - r * Su_vec + w * Sf_vec), 0.0)
        part = jnp.dot(out_c, linw_ref[c * OUT_CH:(c + 1) * OUT_CH, :],
                       preferred_element_type=jnp.float32)
        acc = part if acc is None else acc + part

    out_ref[...] = jnp.maximum(acc + linb_ref[...], 0.0)


def kernel(x, W1, W2, W3, Wgcn, lin_w, lin_b):
    return pl.pallas_call(
        _body,
        out_shape=jax.ShapeDtypeStruct((N, OUT_CH), jnp.float32),
    )(x, W1.T, W2.T, W3.T, Wgcn, lin_w, lin_b.reshape(1, OUT_CH))


# speedup vs baseline: 5.2789x; 2.0422x over previous
import jax
import jax.numpy as jnp
from jax.experimental import pallas as pl
from jax.experimental.pallas import tpu as pltpu

N = 1024
IN_CH = 512
OUT_CH = 512
NUM_CHANNELS = 2


def _softmax_row(row):
    m = jnp.max(row, axis=1, keepdims=True)
    e = jnp.exp(row - m)
    return e / jnp.sum(e, axis=1, keepdims=True)


def _coeff_rows(w1_row, w2_row, w3_row):
    f = _softmax_row(w1_row)
    g = _softmax_row(w2_row)
    h = _softmax_row(w3_row)
    S_g = jnp.sum(g, axis=1, keepdims=True)
    S_f = jnp.sum(f, axis=1, keepdims=True)
    u = S_g * f - f * g
    S_u = jnp.sum(u, axis=1, keepdims=True)
    d = (S_u - u) - g * (S_f - f)
    inv_d = jnp.where(d == 0.0, 0.0, 1.0 / d)
    r = h * inv_d
    w = g * r
    R_ = jnp.sum(r, axis=1, keepdims=True)
    Wt = jnp.sum(w, axis=1, keepdims=True)
    v = u * (R_ - r) - f * (Wt - w)
    S_v = jnp.sum(v, axis=1, keepdims=True)
    deg2 = 1.0 + (S_v - v) - r * (S_u - u) + w * (S_f - f)
    inv2 = jnp.where(deg2 == 0.0, 0.0, 1.0 / deg2)
    beta = inv2 * (1.0 - v + r * u - w * f)
    zero = jnp.zeros_like(f)
    return jnp.concatenate(
        [inv2, inv2 * r, inv2 * w, beta, v, u, f, zero], axis=0)


def _tdot(a, b):
    return jax.lax.dot_general(a, b, (((0,), (0,)), ((), ())),
                               preferred_element_type=jnp.float32)


def _body(x_hbm, w1_ref, w2_ref, w3_ref, wgcn_hbm, linw_hbm, linb_ref,
          out_ref, xv, gv, lv, sx, sg, sl):
    cp_g = pltpu.make_async_copy(wgcn_hbm, gv, sg)
    cp_x = pltpu.make_async_copy(x_hbm, xv, sx)
    cp_l = pltpu.make_async_copy(linw_hbm, lv, sl)
    cp_g.start()
    cp_x.start()
    cp_l.start()

    trows = [
        _coeff_rows(w1_ref[c:c + 1, :], w2_ref[c:c + 1, :], w3_ref[c:c + 1, :])
        for c in range(NUM_CHANNELS)
    ]
    eye8 = jnp.eye(8, dtype=jnp.float32)
    tcols = [_tdot(t, eye8) for t in trows]

    cp_g.wait()
    cp_x.wait()
    Xw = jnp.dot(xv[...], gv[...], preferred_element_type=jnp.float32)

    outs = []
    for c in range(NUM_CHANNELS):
        R8 = jnp.dot(trows[c], Xw, preferred_element_type=jnp.float32)
        M = jnp.concatenate(
            [R8[4:5, :], -R8[5:6, :], R8[6:7, :],
             jnp.zeros((5, OUT_CH), dtype=jnp.float32)], axis=0)
        corr = jnp.dot(tcols[c], M, preferred_element_type=jnp.float32)
        beta = tcols[c][:, 3:4]
        outs.append(jnp.maximum(beta * Xw + corr, 0.0))

    cp_l.wait()
    acc = jnp.dot(outs[0], lv[0:OUT_CH, :], preferred_element_type=jnp.float32)
    acc = acc + jnp.dot(outs[1], lv[OUT_CH:2 * OUT_CH, :],
                        preferred_element_type=jnp.float32)
    out_ref[...] = jnp.maximum(acc + linb_ref[...], 0.0)


def kernel(x, W1, W2, W3, Wgcn, lin_w, lin_b):
    return pl.pallas_call(
        _body,
        in_specs=[
            pl.BlockSpec(memory_space=pltpu.HBM),
            pl.BlockSpec(memory_space=pltpu.VMEM),
            pl.BlockSpec(memory_space=pltpu.VMEM),
            pl.BlockSpec(memory_space=pltpu.VMEM),
            pl.BlockSpec(memory_space=pltpu.HBM),
            pl.BlockSpec(memory_space=pltpu.HBM),
            pl.BlockSpec(memory_space=pltpu.VMEM),
        ],
        out_specs=pl.BlockSpec(memory_space=pltpu.VMEM),
        out_shape=jax.ShapeDtypeStruct((N, OUT_CH), jnp.float32),
        scratch_shapes=[
            pltpu.VMEM((N, IN_CH), jnp.float32),
            pltpu.VMEM((IN_CH, OUT_CH), jnp.float32),
            pltpu.VMEM((OUT_CH * NUM_CHANNELS, OUT_CH), jnp.float32),
            pltpu.SemaphoreType.DMA,
            pltpu.SemaphoreType.DMA,
            pltpu.SemaphoreType.DMA,
        ],
    )(x, W1, W2, W3, Wgcn, lin_w, lin_b.reshape(1, OUT_CH))
